# Initial kernel scaffold; baseline (speedup 1.0000x reference)
import jax
import jax.numpy as jnp
from jax.experimental import pallas as pl


def kernel(input, sizes):
    raise NotImplementedError("write your pallas kernel here")


# speedup vs baseline: 6.1119x; 6.1119x over previous
import functools

import jax
import jax.numpy as jnp
from jax import lax
from jax.experimental import pallas as pl
from jax.experimental.pallas import tpu as pltpu
from jax.experimental.pallas import tpu_sc as plsc

NROWS = 32768
NCOLS = 512
NSEG = 16
HALF = NCOLS // 2
NV = HALF // 16
RBLK = 256

_NEG_INF = float("-inf")


def _dmax_body(x_hbm, starts_hbm, ends_hbm, out_hbm, sv, ev, buf_v, res_v):
    half = lax.axis_index("c")
    seg = lax.axis_index("s")

    pltpu.sync_copy(starts_hbm, sv)
    pltpu.sync_copy(ends_hbm, ev)
    starts_v = sv[...]
    ends_v = ev[...]
    lane = lax.broadcasted_iota(jnp.int32, (16,), 0)
    selm = lane == seg
    a = jnp.sum(jnp.where(selm, starts_v, 0))
    b = jnp.sum(jnp.where(selm, ends_v, 0))
    a8 = (a // 8) * 8
    nblk = (b - a8 + RBLK - 1) // RBLK
    col0 = pl.multiple_of(half * HALF, 128)

    acc0 = tuple(jnp.full((16,), _NEG_INF, jnp.float32) for _ in range(NV))

    def blk_body(k, acc):
        s = jnp.minimum(a8 + k * RBLK, NROWS - RBLK)
        s = pl.multiple_of(s, 8)
        pltpu.sync_copy(x_hbm.at[pl.ds(s, RBLK), pl.ds(col0, HALF)], buf_v)

        def row_body(r, acc2):
            g = s + r
            ok = (g >= a) & (g < b)
            new = []
            for c in range(NV):
                v = buf_v[r, pl.ds(c * 16, 16)]
                v = jnp.where(ok, v, _NEG_INF)
                new.append(jnp.maximum(acc2[c], v))
            return tuple(new)

        return lax.fori_loop(0, RBLK, row_body, acc)

    acc = lax.fori_loop(0, nblk, blk_body, acc0)
    for c in range(NV):
        res_v[pl.ds(c * 16, 16)] = acc[c]
    off = pl.multiple_of(seg * NCOLS + col0, 256)
    pltpu.sync_copy(res_v, out_hbm.at[pl.ds(off, HALF)])


@jax.jit
def _dmax(x, starts, ends):
    mesh = plsc.VectorSubcoreMesh(core_axis_name="c", subcore_axis_name="s")
    return pl.kernel(
        _dmax_body,
        out_type=jax.ShapeDtypeStruct((NSEG * NCOLS,), jnp.float32),
        mesh=mesh,
        compiler_params=pltpu.CompilerParams(needs_layout_passes=False),
        scratch_types=[
            pltpu.VMEM((16,), jnp.int32),
            pltpu.VMEM((16,), jnp.int32),
            pltpu.VMEM((RBLK, HALF), jnp.float32),
            pltpu.VMEM((HALF,), jnp.float32),
        ],
    )(x, starts, ends)


def kernel(input, sizes):
    sizes = sizes.astype(jnp.int32)
    ends = jnp.cumsum(sizes, dtype=jnp.int32)
    starts = ends - sizes
    return _dmax(input, starts, ends).reshape(NSEG, NCOLS)


# speedup vs baseline: 7.7273x; 1.2643x over previous
import functools

import jax
import jax.numpy as jnp
from jax import lax
from jax.experimental import pallas as pl
from jax.experimental.pallas import tpu as pltpu
from jax.experimental.pallas import tpu_sc as plsc

NROWS = 32768
NCOLS = 512
NSEG = 16
HALF = NCOLS // 2
NV = HALF // 16
RBLK = 128
UNROLL = 4

_NEG_INF = float("-inf")


def _dmax_body(x_hbm, starts_hbm, ends_hbm, out_hbm, sv, ev, buf_v, res_v, sem):
    half = lax.axis_index("c")
    seg = lax.axis_index("s")

    pltpu.sync_copy(starts_hbm, sv)
    pltpu.sync_copy(ends_hbm, ev)
    starts_v = sv[...]
    ends_v = ev[...]
    lane = lax.broadcasted_iota(jnp.int32, (16,), 0)
    selm = lane == seg
    a = jnp.sum(jnp.where(selm, starts_v, 0))
    b = jnp.sum(jnp.where(selm, ends_v, 0))
    a8 = (a // 8) * 8
    nblk = (b - a8 + RBLK - 1) // RBLK
    col0 = pl.multiple_of(half * HALF, 128)

    acc0 = tuple(jnp.full((16,), _NEG_INF, jnp.float32) for _ in range(NV))

    def row_of(k):
        s = jnp.minimum(a8 + k * RBLK, NROWS - RBLK)
        return pl.multiple_of(s, 8)

    def dma(k):
        s = row_of(k)
        return pltpu.make_async_copy(
            x_hbm.at[pl.ds(s, RBLK), pl.ds(col0, HALF)],
            buf_v.at[k % 2],
            sem.at[k % 2],
        )

    dma(0).start()

    def blk_body(k, acc):
        dma(k).wait()

        @pl.when(k + 1 < nblk)
        def _():
            dma(k + 1).start()

        s = row_of(k)
        k2 = k % 2

        def row_body(r0, acc2):
            r = r0 * UNROLL
            cur = list(acc2)
            for j in range(UNROLL):
                g = s + r + j
                ok = (g >= a) & (g < b)
                for c in range(NV):
                    v = buf_v[k2, r + j, pl.ds(c * 16, 16)]
                    v = jnp.where(ok, v, _NEG_INF)
                    cur[c] = jnp.maximum(cur[c], v)
            return tuple(cur)

        return lax.fori_loop(0, RBLK // UNROLL, row_body, acc)

    acc = lax.fori_loop(0, nblk, blk_body, acc0)
    for c in range(NV):
        res_v[pl.ds(c * 16, 16)] = acc[c]
    off = pl.multiple_of(seg * NCOLS + col0, 256)
    pltpu.sync_copy(res_v, out_hbm.at[pl.ds(off, HALF)])


@jax.jit
def _dmax(x, starts, ends):
    mesh = plsc.VectorSubcoreMesh(core_axis_name="c", subcore_axis_name="s")
    return pl.kernel(
        _dmax_body,
        out_type=jax.ShapeDtypeStruct((NSEG * NCOLS,), jnp.float32),
        mesh=mesh,
        compiler_params=pltpu.CompilerParams(needs_layout_passes=False),
        scratch_types=[
            pltpu.VMEM((16,), jnp.int32),
            pltpu.VMEM((16,), jnp.int32),
            pltpu.VMEM((2, RBLK, HALF), jnp.float32),
            pltpu.VMEM((HALF,), jnp.float32),
            pltpu.SemaphoreType.DMA((2,)),
        ],
    )(x, starts, ends)


def kernel(input, sizes):
    sizes = sizes.astype(jnp.int32)
    ends = jnp.cumsum(sizes, dtype=jnp.int32)
    starts = ends - sizes
    return _dmax(input, starts, ends).reshape(NSEG, NCOLS)


# speedup vs baseline: 8.7568x; 1.1332x over previous
import functools

import jax
import jax.numpy as jnp
from jax import lax
from jax.experimental import pallas as pl
from jax.experimental.pallas import tpu as pltpu
from jax.experimental.pallas import tpu_sc as plsc

NROWS = 32768
NCOLS = 512
NSEG = 16
HALF = NCOLS // 2
NV = HALF // 16
RBLK = 128
UNROLL = 4

_NEG_INF = float("-inf")


def _dmax_body(x_hbm, starts_hbm, ends_hbm, out_hbm,
               sv, ev, buf_v, part_v, mbuf_v, res_v, sem, spmem):
    half = lax.axis_index("c")
    tile = lax.axis_index("s")

    pltpu.sync_copy(starts_hbm, sv)
    pltpu.sync_copy(ends_hbm, ev)
    starts_v = sv[...]
    ends_v = ev[...]
    lane = lax.broadcasted_iota(jnp.int32, (16,), 0)

    total = jnp.sum(jnp.where(lane == NSEG - 1, ends_v, 0))
    q = (total + NSEG - 1) // NSEG
    lo = tile * q
    hi = jnp.minimum(lo + q, total)

    col0 = pl.multiple_of(half * HALF, 128)

    def init_body(r, _):
        for c in range(NV):
            part_v[r, pl.ds(c * 16, 16)] = jnp.full((16,), _NEG_INF, jnp.float32)
        return 0

    lax.fori_loop(0, NSEG, init_body, 0)

    sid_lo = jnp.sum((ends_v <= lo).astype(jnp.int32))
    sid_hi = jnp.sum((starts_v < hi).astype(jnp.int32))

    acc0 = tuple(jnp.full((16,), _NEG_INF, jnp.float32) for _ in range(NV))

    def seg_body(i, _):
        selm = lane == i
        ai = jnp.sum(jnp.where(selm, starts_v, 0))
        bi = jnp.sum(jnp.where(selm, ends_v, 0))
        a2 = jnp.maximum(ai, lo)
        b2 = jnp.minimum(bi, hi)
        a8 = (a2 // 8) * 8
        nblk = (b2 - a8 + RBLK - 1) // RBLK

        def row_of(k):
            return pl.multiple_of(jnp.minimum(a8 + k * RBLK, NROWS - RBLK), 8)

        def dma(k):
            return pltpu.make_async_copy(
                x_hbm.at[pl.ds(row_of(k), RBLK), pl.ds(col0, HALF)],
                buf_v.at[k % 2],
                sem.at[k % 2],
            )

        dma(0).start()

        def blk_body(k, acc):
            dma(k).wait()

            @pl.when(k + 1 < nblk)
            def _():
                dma(k + 1).start()

            s = row_of(k)
            k2 = k % 2

            def row_body(r0, acc2):
                r = r0 * UNROLL
                cur = list(acc2)
                for j in range(UNROLL):
                    g = s + r + j
                    ok = (g >= a2) & (g < b2)
                    for c in range(NV):
                        v = buf_v[k2, r + j, pl.ds(c * 16, 16)]
                        cur[c] = jnp.maximum(cur[c], jnp.where(ok, v, _NEG_INF))
                return tuple(cur)

            return lax.fori_loop(0, RBLK // UNROLL, row_body, acc)

        acc = lax.fori_loop(0, nblk, blk_body, acc0)
        for c in range(NV):
            part_v[i, pl.ds(c * 16, 16)] = acc[c]
        return 0

    lax.fori_loop(sid_lo, sid_hi, seg_body, 0)

    for i in range(NSEG):
        pltpu.sync_copy(part_v.at[i], spmem.at[half, i, tile])
    plsc.subcore_barrier()

    pltpu.sync_copy(spmem.at[half, tile], mbuf_v)

    def mrg_body(w, acc):
        return tuple(
            jnp.maximum(acc[c], mbuf_v[w, pl.ds(c * 16, 16)]) for c in range(NV)
        )

    macc = lax.fori_loop(0, NSEG, mrg_body, acc0)
    for c in range(NV):
        res_v[pl.ds(c * 16, 16)] = macc[c]
    off = pl.multiple_of(tile * NCOLS + col0, 256)
    pltpu.sync_copy(res_v, out_hbm.at[pl.ds(off, HALF)])


@jax.jit
def _dmax(x, starts, ends):
    mesh = plsc.VectorSubcoreMesh(core_axis_name="c", subcore_axis_name="s")
    return pl.kernel(
        _dmax_body,
        out_type=jax.ShapeDtypeStruct((NSEG * NCOLS,), jnp.float32),
        mesh=mesh,
        compiler_params=pltpu.CompilerParams(needs_layout_passes=False),
        scratch_types=[
            pltpu.VMEM((16,), jnp.int32),
            pltpu.VMEM((16,), jnp.int32),
            pltpu.VMEM((2, RBLK, HALF), jnp.float32),
            pltpu.VMEM((NSEG, HALF), jnp.float32),
            pltpu.VMEM((NSEG, HALF), jnp.float32),
            pltpu.VMEM((HALF,), jnp.float32),
            pltpu.SemaphoreType.DMA((2,)),
            pltpu.VMEM_SHARED((2, NSEG, NSEG, HALF), jnp.float32),
        ],
    )(x, starts, ends)


def kernel(input, sizes):
    sizes = sizes.astype(jnp.int32)
    ends = jnp.cumsum(sizes, dtype=jnp.int32)
    starts = ends - sizes
    return _dmax(input, starts, ends).reshape(NSEG, NCOLS)


# speedup vs baseline: 9.0562x; 1.0342x over previous
import functools

import jax
import jax.numpy as jnp
from jax import lax
from jax.experimental import pallas as pl
from jax.experimental.pallas import tpu as pltpu
from jax.experimental.pallas import tpu_sc as plsc

NROWS = 32768
NCOLS = 512
NSEG = 16
HALF = NCOLS // 2
NV = HALF // 16
RBLK = 128
NBUF = 3
UNROLL = 4

_NEG_INF = float("-inf")


def _dmax_body(x_hbm, sizes_hbm, out_hbm,
               sv, buf_v, part_v, mbuf_v, res_v, sem, spmem):
    half = lax.axis_index("c")
    tile = lax.axis_index("s")

    pltpu.sync_copy(sizes_hbm, sv)
    sizes_v = sv[...]
    ends_v = plsc.cumsum(sizes_v)
    starts_v = ends_v - sizes_v
    lane = lax.broadcasted_iota(jnp.int32, (16,), 0)

    total = jnp.sum(jnp.where(lane == NSEG - 1, ends_v, 0))
    q = (total + NSEG - 1) // NSEG
    lo = tile * q
    hi = jnp.minimum(lo + q, total)

    col0 = pl.multiple_of(half * HALF, 128)

    def init_body(r, _):
        for c in range(NV):
            part_v[r, pl.ds(c * 16, 16)] = jnp.full((16,), _NEG_INF, jnp.float32)
        return 0

    lax.fori_loop(0, NSEG, init_body, 0)

    sid_lo = jnp.sum((ends_v <= lo).astype(jnp.int32))
    sid_hi = jnp.sum((starts_v < hi).astype(jnp.int32))

    acc0 = tuple(jnp.full((16,), _NEG_INF, jnp.float32) for _ in range(NV))

    def seg_body(i, _):
        selm = lane == i
        ai = jnp.sum(jnp.where(selm, starts_v, 0))
        bi = jnp.sum(jnp.where(selm, ends_v, 0))
        a2 = jnp.maximum(ai, lo)
        b2 = jnp.minimum(bi, hi)
        a8 = (a2 // 8) * 8
        nblk = (b2 - a8 + RBLK - 1) // RBLK

        def row_of(k):
            return pl.multiple_of(jnp.minimum(a8 + k * RBLK, NROWS - RBLK), 8)

        def dma(k):
            return pltpu.make_async_copy(
                x_hbm.at[pl.ds(row_of(k), RBLK), pl.ds(col0, HALF)],
                buf_v.at[k % NBUF],
                sem.at[k % NBUF],
            )

        dma(0).start()
        for p in range(1, NBUF - 1):
            @pl.when(p < nblk)
            def _(p=p):
                dma(p).start()

        def blk_body(k, acc):
            dma(k).wait()

            @pl.when(k + NBUF - 1 < nblk)
            def _():
                dma(k + NBUF - 1).start()

            s = row_of(k)
            k2 = k % NBUF

            def row_body(r0, acc2):
                r = r0 * UNROLL
                cur = list(acc2)
                for j in range(UNROLL):
                    g = s + r + j
                    ok = (g >= a2) & (g < b2)
                    for c in range(NV):
                        v = buf_v[k2, r + j, pl.ds(c * 16, 16)]
                        cur[c] = jnp.maximum(cur[c], jnp.where(ok, v, _NEG_INF))
                return tuple(cur)

            return lax.fori_loop(0, RBLK // UNROLL, row_body, acc)

        acc = lax.fori_loop(0, nblk, blk_body, acc0)
        for c in range(NV):
            part_v[i, pl.ds(c * 16, 16)] = acc[c]
        return 0

    lax.fori_loop(sid_lo, sid_hi, seg_body, 0)

    for i in range(NSEG):
        pltpu.sync_copy(part_v.at[i], spmem.at[half, i, tile])
    plsc.subcore_barrier()

    pltpu.sync_copy(spmem.at[half, tile], mbuf_v)

    def mrg_body(w, acc):
        return tuple(
            jnp.maximum(acc[c], mbuf_v[w, pl.ds(c * 16, 16)]) for c in range(NV)
        )

    macc = lax.fori_loop(0, NSEG, mrg_body, acc0)
    for c in range(NV):
        res_v[pl.ds(c * 16, 16)] = macc[c]
    off = pl.multiple_of(tile * NCOLS + col0, 256)
    pltpu.sync_copy(res_v, out_hbm.at[pl.ds(off, HALF)])


@jax.jit
def _dmax(x, sizes):
    mesh = plsc.VectorSubcoreMesh(core_axis_name="c", subcore_axis_name="s")
    return pl.kernel(
        _dmax_body,
        out_type=jax.ShapeDtypeStruct((NSEG * NCOLS,), jnp.float32),
        mesh=mesh,
        compiler_params=pltpu.CompilerParams(needs_layout_passes=False),
        scratch_types=[
            pltpu.VMEM((16,), jnp.int32),
            pltpu.VMEM((NBUF, RBLK, HALF), jnp.float32),
            pltpu.VMEM((NSEG, HALF), jnp.float32),
            pltpu.VMEM((NSEG, HALF), jnp.float32),
            pltpu.VMEM((HALF,), jnp.float32),
            pltpu.SemaphoreType.DMA((NBUF,)),
            pltpu.VMEM_SHARED((2, NSEG, NSEG, HALF), jnp.float32),
        ],
    )(x, sizes)


def kernel(input, sizes):
    return _dmax(input, sizes.astype(jnp.int32)).reshape(NSEG, NCOLS)
